# baseline (speedup 1.0000x reference)
from functools import partial

import jax
import jax.numpy as jnp
from jax.experimental import pallas as pl
from jax.experimental.pallas import tpu as pltpu

_EPS = 1e-6


def _gem_rows_kernel(p_ref, x_ref, o_ref, *, inv_hw):
    p = p_ref[0]
    x = x_ref[...]
    xc = jnp.maximum(x, _EPS)
    t = jnp.exp(p * jnp.log(xc))
    s = jnp.sum(t, axis=-1, keepdims=True) * inv_hw
    o_ref[...] = jnp.exp(jnp.log(s) / p)


def _gem_rows(x2, p_arr, block_rows):
    rows, hw = x2.shape
    grid = rows // block_rows
    return pl.pallas_call(
        partial(_gem_rows_kernel, inv_hw=1.0 / hw),
        out_shape=jax.ShapeDtypeStruct((rows, 1), jnp.float32),
        grid=(grid,),
        in_specs=[
            pl.BlockSpec(memory_space=pltpu.MemorySpace.SMEM),
            pl.BlockSpec((block_rows, hw), lambda i: (i, 0)),
        ],
        out_specs=pl.BlockSpec((block_rows, 1), lambda i: (i, 0)),
        compiler_params=pltpu.CompilerParams(
            dimension_semantics=("parallel",),
        ),
        cost_estimate=pl.CostEstimate(
            flops=5 * rows * hw,
            transcendentals=2 * rows * hw,
            bytes_accessed=rows * hw * 4 + rows * 4,
        ),
    )(p_arr, x2)


def kernel(x, p):
    N, C, H, W = x.shape
    hw = H * W
    rows = N * C
    x2 = x.reshape(rows, hw).astype(jnp.float32)
    p_arr = jnp.asarray(p, jnp.float32).reshape(1)

    block_rows = 2048
    while rows % block_rows:
        block_rows //= 2

    out = _gem_rows(x2, p_arr, block_rows)
    return out.reshape(N, C, 1, 1).astype(x.dtype)


# speedup vs baseline: 1.9398x; 1.9398x over previous
from functools import partial

import jax
import jax.numpy as jnp
from jax.experimental import pallas as pl
from jax.experimental.pallas import tpu as pltpu

_EPS = 1e-6


def _gem_kernel(p_ref, x_ref, o_ref, *, inv_hw):
    p = p_ref[0]
    xc = jnp.maximum(x_ref[...], _EPS)
    t = jnp.exp2(p * jnp.log2(xc))
    s = jnp.sum(t, axis=-1, keepdims=True) * inv_hw
    o_ref[...] = jnp.exp2(jnp.log2(s) / p)


def _gem(x3, p_arr, nb):
    N, C, hw = x3.shape
    grid = N // nb
    return pl.pallas_call(
        partial(_gem_kernel, inv_hw=1.0 / hw),
        out_shape=jax.ShapeDtypeStruct((N, C, 1), jnp.float32),
        grid=(grid,),
        in_specs=[
            pl.BlockSpec(memory_space=pltpu.MemorySpace.SMEM),
            pl.BlockSpec((nb, C, hw), lambda i: (i, 0, 0)),
        ],
        out_specs=pl.BlockSpec((nb, C, 1), lambda i: (i, 0, 0)),
        compiler_params=pltpu.CompilerParams(
            dimension_semantics=("parallel",),
        ),
        cost_estimate=pl.CostEstimate(
            flops=5 * N * C * hw,
            transcendentals=2 * N * C * hw,
            bytes_accessed=N * C * hw * 4 + N * C * 4,
        ),
    )(p_arr, x3)


def kernel(x, p):
    N, C, H, W = x.shape
    hw = H * W
    x3 = x.reshape(N, C, hw).astype(jnp.float32)
    p_arr = jnp.asarray(p, jnp.float32).reshape(1)

    nb = 4
    while N % nb:
        nb //= 2

    out = _gem(x3, p_arr, nb)
    return out.reshape(N, C, 1, 1).astype(x.dtype)


# speedup vs baseline: 2.1495x; 1.1081x over previous
from functools import partial

import jax
import jax.numpy as jnp
from jax.experimental import pallas as pl
from jax.experimental.pallas import tpu as pltpu

_EPS = 1e-6


def _gem_kernel(p_ref, x_ref, o_ref, *, inv_hw):
    p = p_ref[0]
    xc = jnp.maximum(x_ref[...], _EPS)
    t = jnp.exp2(p * jnp.log2(xc))
    s = jnp.sum(t, axis=-1, keepdims=True) * inv_hw
    o_ref[...] = jnp.exp2(jnp.log2(s) / p)


def _gem(x3, p_arr, nb):
    N, C, hw = x3.shape
    grid = N // nb
    return pl.pallas_call(
        partial(_gem_kernel, inv_hw=1.0 / hw),
        out_shape=jax.ShapeDtypeStruct((N, C, 1), jnp.float32),
        grid=(grid,),
        in_specs=[
            pl.BlockSpec(memory_space=pltpu.MemorySpace.SMEM),
            pl.BlockSpec((nb, C, hw), lambda i: (i, 0, 0)),
        ],
        out_specs=pl.BlockSpec((nb, C, 1), lambda i: (i, 0, 0)),
        compiler_params=pltpu.CompilerParams(
            dimension_semantics=("parallel",),
        ),
        cost_estimate=pl.CostEstimate(
            flops=5 * N * C * hw,
            transcendentals=2 * N * C * hw,
            bytes_accessed=N * C * hw * 4 + N * C * 4,
        ),
    )(p_arr, x3)


def kernel(x, p):
    N, C, H, W = x.shape
    hw = H * W
    x3 = x.reshape(N, C, hw).astype(jnp.float32)
    p_arr = jnp.asarray(p, jnp.float32).reshape(1)

    nb = 16
    while N % nb:
        nb //= 2

    out = _gem(x3, p_arr, nb)
    return out.reshape(N, C, 1, 1).astype(x.dtype)
